# baseline (speedup 1.0000x reference)
import functools

import jax
import jax.numpy as jnp
from jax.experimental import pallas as pl
from jax.experimental.pallas import tpu as pltpu

MARGIN = 0.3
K_POS = 8
K_NEG = 64
EPS = 1e-8

BLOCK = 8192
LANES = 128


def _float_keys(vals, valid):
    bits = jax.lax.bitcast_convert_type(vals, jnp.uint32)
    neg = bits >= jnp.uint32(0x80000000)
    key = jnp.where(neg, ~bits, bits | jnp.uint32(0x80000000))
    return jnp.where(valid, key, jnp.uint32(0))


def _key_to_float(key):
    neg = key < jnp.uint32(0x80000000)
    bits = jnp.where(neg, ~key, key ^ jnp.uint32(0x80000000))
    return jax.lax.bitcast_convert_type(bits, jnp.float32)


def _kth_largest(vals, valid, k):
    keys = _float_keys(vals, valid)
    prefix = jnp.uint32(0)
    for b in range(30, -1, -2):
        c3 = prefix | jnp.uint32(3 << b)
        c2 = prefix | jnp.uint32(2 << b)
        c1 = prefix | jnp.uint32(1 << b)
        n3 = jnp.sum((keys >= c3).astype(jnp.int32))
        n2 = jnp.sum((keys >= c2).astype(jnp.int32))
        n1 = jnp.sum((keys >= c1).astype(jnp.int32))
        prefix = jnp.where(
            n3 >= k, c3, jnp.where(n2 >= k, c2, jnp.where(n1 >= k, c1, prefix)))
    return _key_to_float(prefix)


def _loss_kernel(img_ref, anchor_ref, out_ref, cos_scratch, *, n_rows):
    i = pl.program_id(0)
    nblocks = pl.num_programs(0)

    x = img_ref[...]
    a = anchor_ref[0:1, :]
    dot = jax.lax.dot_general(
        x, a, (((1,), (1,)), ((), ())),
        preferred_element_type=jnp.float32)
    xb = x.astype(jnp.bfloat16)
    ones = jnp.ones((a.shape[1], 1), jnp.bfloat16)
    sq = jax.lax.dot_general(
        xb * xb, ones, (((1,), (0,)), ((), ())),
        preferred_element_type=jnp.float32)
    na = jnp.sqrt(jnp.sum(a * a))
    rows = BLOCK // LANES
    dot_r = dot.reshape(rows, LANES)
    sq_r = sq.reshape(rows, LANES)
    denom = jnp.maximum(na * jnp.sqrt(sq_r), EPS)
    cos_scratch[pl.ds(i * rows, rows), :] = dot_r / denom

    @pl.when(i == nblocks - 1)
    def _finish():
        cosv = cos_scratch[...]
        r_idx = jax.lax.broadcasted_iota(jnp.int32, cosv.shape, 0)
        c_idx = jax.lax.broadcasted_iota(jnp.int32, cosv.shape, 1)
        pos = r_idx * LANES + c_idx

        lane = jax.lax.broadcasted_iota(jnp.int32, (1, LANES), 1)
        vp = (lane >= 1) & (lane <= 16)
        d_row = jnp.where(vp, 1.0 - cosv[0:1, :], -3.0e38)
        d_col = jnp.transpose(d_row)
        l_col = jnp.transpose(lane)
        beats = (d_row > d_col) | ((d_row == d_col) & (lane < l_col))
        rank = jnp.sum(beats.astype(jnp.int32), axis=1, keepdims=True)
        sel8 = jnp.transpose(vp) & (rank < K_POS)
        mean_p = jnp.sum(jnp.where(sel8, d_col, 0.0)) / K_POS

        valid_n = (pos >= 17) & (pos < n_rows)
        t64 = _kth_largest(cosv, valid_n, K_NEG)
        c = mean_p + MARGIN
        h = jnp.maximum(c - (1.0 - cosv), 0.0)
        gtn = valid_n & (cosv > t64)
        gn = jnp.sum(gtn.astype(jnp.int32))
        sh = jnp.sum(jnp.where(gtn, h, 0.0))
        ht = jnp.maximum(c - (1.0 - t64), 0.0)
        loss = (sh + (K_NEG - gn).astype(jnp.float32) * ht) / K_NEG
        out_ref[...] = jnp.reshape(loss, (1, 1))


def kernel(img):
    n, d = img.shape
    nblocks = pl.cdiv(n, BLOCK)
    scratch_rows = nblocks * BLOCK // LANES
    out = pl.pallas_call(
        functools.partial(_loss_kernel, n_rows=n),
        grid=(nblocks,),
        in_specs=[
            pl.BlockSpec((BLOCK, d), lambda i: (i, 0)),
            pl.BlockSpec((8, d), lambda i: (0, 0)),
        ],
        out_specs=pl.BlockSpec((1, 1), lambda i: (0, 0)),
        out_shape=jax.ShapeDtypeStruct((1, 1), jnp.float32),
        scratch_shapes=[pltpu.VMEM((scratch_rows, LANES), jnp.float32)],
    )(img, img)
    return out[0, 0]


# speedup vs baseline: 1.1251x; 1.1251x over previous
import functools

import jax
import jax.numpy as jnp
from jax.experimental import pallas as pl
from jax.experimental.pallas import tpu as pltpu

MARGIN = 0.3
K_POS = 8
K_NEG = 64
EPS = 1e-8

BLOCK = 8192
LANES = 128


def _float_keys(vals, valid):
    bits = jax.lax.bitcast_convert_type(vals, jnp.uint32)
    neg = bits >= jnp.uint32(0x80000000)
    key = jnp.where(neg, ~bits, bits | jnp.uint32(0x80000000))
    return jnp.where(valid, key, jnp.uint32(0))


def _key_to_float(key):
    neg = key < jnp.uint32(0x80000000)
    bits = jnp.where(neg, ~key, key ^ jnp.uint32(0x80000000))
    return jax.lax.bitcast_convert_type(bits, jnp.float32)


def _kth_largest(vals, valid, k):
    keys = _float_keys(vals, valid)
    prefix = jnp.uint32(0)
    for b in range(30, -1, -2):
        c3 = prefix | jnp.uint32(3 << b)
        c2 = prefix | jnp.uint32(2 << b)
        c1 = prefix | jnp.uint32(1 << b)
        n3 = jnp.sum((keys >= c3).astype(jnp.int32))
        n2 = jnp.sum((keys >= c2).astype(jnp.int32))
        n1 = jnp.sum((keys >= c1).astype(jnp.int32))
        prefix = jnp.where(
            n3 >= k, c3, jnp.where(n2 >= k, c2, jnp.where(n1 >= k, c1, prefix)))
    return _key_to_float(prefix)


def _loss_kernel(img_ref, anchor_ref, out_ref, cos_scratch, *, n_rows):
    i = pl.program_id(0)
    nblocks = pl.num_programs(0)

    x = img_ref[...]
    a = anchor_ref[0:1, :]
    xb = x.astype(jnp.bfloat16)
    ab = jnp.transpose(a.astype(jnp.bfloat16))
    dot = jax.lax.dot_general(
        xb, ab, (((1,), (0,)), ((), ())),
        preferred_element_type=jnp.float32)
    ones = jnp.ones((a.shape[1], 1), jnp.bfloat16)
    sq = jax.lax.dot_general(
        xb * xb, ones, (((1,), (0,)), ((), ())),
        preferred_element_type=jnp.float32)
    na = jnp.sqrt(jnp.sum(a * a))
    rows = BLOCK // LANES
    dot_r = dot.reshape(rows, LANES)
    sq_r = sq.reshape(rows, LANES)
    denom = jnp.maximum(na * jnp.sqrt(sq_r), EPS)
    cos_scratch[pl.ds(i * rows, rows), :] = dot_r / denom

    @pl.when(i == nblocks - 1)
    def _finish():
        cosv = cos_scratch[...]
        r_idx = jax.lax.broadcasted_iota(jnp.int32, cosv.shape, 0)
        c_idx = jax.lax.broadcasted_iota(jnp.int32, cosv.shape, 1)
        pos = r_idx * LANES + c_idx

        lane = jax.lax.broadcasted_iota(jnp.int32, (1, LANES), 1)
        vp = (lane >= 1) & (lane <= 16)
        d_row = jnp.where(vp, 1.0 - cosv[0:1, :], -3.0e38)
        d_col = jnp.transpose(d_row)
        l_col = jnp.transpose(lane)
        beats = (d_row > d_col) | ((d_row == d_col) & (lane < l_col))
        rank = jnp.sum(beats.astype(jnp.int32), axis=1, keepdims=True)
        sel8 = jnp.transpose(vp) & (rank < K_POS)
        mean_p = jnp.sum(jnp.where(sel8, d_col, 0.0)) / K_POS

        valid_n = (pos >= 17) & (pos < n_rows)
        t64 = _kth_largest(cosv, valid_n, K_NEG)
        c = mean_p + MARGIN
        h = jnp.maximum(c - (1.0 - cosv), 0.0)
        gtn = valid_n & (cosv > t64)
        gn = jnp.sum(gtn.astype(jnp.int32))
        sh = jnp.sum(jnp.where(gtn, h, 0.0))
        ht = jnp.maximum(c - (1.0 - t64), 0.0)
        loss = (sh + (K_NEG - gn).astype(jnp.float32) * ht) / K_NEG
        out_ref[...] = jnp.reshape(loss, (1, 1))


def kernel(img):
    n, d = img.shape
    nblocks = pl.cdiv(n, BLOCK)
    scratch_rows = nblocks * BLOCK // LANES
    out = pl.pallas_call(
        functools.partial(_loss_kernel, n_rows=n),
        grid=(nblocks,),
        in_specs=[
            pl.BlockSpec((BLOCK, d), lambda i: (i, 0)),
            pl.BlockSpec((8, d), lambda i: (0, 0)),
        ],
        out_specs=pl.BlockSpec((1, 1), lambda i: (0, 0)),
        out_shape=jax.ShapeDtypeStruct((1, 1), jnp.float32),
        scratch_shapes=[pltpu.VMEM((scratch_rows, LANES), jnp.float32)],
    )(img, img)
    return out[0, 0]
